# baseline (speedup 1.0000x reference)
import functools

import jax
import jax.numpy as jnp
from jax.experimental import pallas as pl
from jax.experimental.pallas import tpu as pltpu


def _pool_body(bb, rpb, x_ref, wlo_ref, whi_ref, o_ref):
    x = x_ref[...]
    rows = bb * rpb
    t = x.T
    hlo = jnp.sum(t[0:64, :], axis=0)
    hhi = jnp.sum(t[64:128, :], axis=0)
    prod = wlo_ref[...] * hlo[None, :] + whi_ref[...] * hhi[None, :]
    cols = []
    for b in range(bb):
        seg = prod[:, b * rpb:(b + 1) * rpb]
        fold = seg[:, 0:128]
        for j in range(1, rpb // 128):
            fold = fold + seg[:, j * 128:(j + 1) * 128]
        width = 128
        while width > 1:
            h = width // 2
            fold = fold[:, 0:h] + fold[:, h:width]
            width = h
        cols.append(fold)
    blk = jnp.concatenate(cols, axis=1)
    i = pl.program_id(0)
    o_ref[pl.ds(i * bb, bb), :] = blk.T


def _routing_body(l_ref, coef_ref, g_ref, loss_ref):
    logits = l_ref[...]
    b_dim, m_dim = logits.shape
    col = jax.lax.broadcasted_iota(jnp.int32, (b_dim, m_dim), 1)
    big = jnp.int32(m_dim)

    m1 = jnp.max(logits, axis=1, keepdims=True)
    idx1 = jnp.min(jnp.where(logits == m1, col, big), axis=1, keepdims=True)
    masked = jnp.where(col == idx1, -jnp.inf, logits)
    m2 = jnp.max(masked, axis=1, keepdims=True)
    idx2 = jnp.min(jnp.where(masked == m2, col, big), axis=1, keepdims=True)

    e = jnp.exp(m2 - m1)
    denom = 1.0 + e
    g1 = 1.0 / denom
    g2 = e / denom
    gates = (jnp.where(col == idx1, g1, 0.0)
             + jnp.where(col == idx2, g2, 0.0))
    g_ref[...] = gates

    imp = jnp.sum(gates, axis=0, keepdims=True)
    load = jnp.sum((gates > 0.0).astype(jnp.float32), axis=0,
                   keepdims=True)

    def cv_sq(v):
        mean = jnp.sum(v) * jnp.float32(1.0 / m_dim)
        var = jnp.sum((v - mean) ** 2) * jnp.float32(1.0 / (m_dim - 1))
        return var / (mean * mean + jnp.float32(1e-10))

    loss_ref[0, 0] = (cv_sq(imp) + cv_sq(load)) * coef_ref[0]


def kernel(feats, w_gate, w_noise, loss_coef=0.01, noise_epsilon=0.01):
    B, C, H, W = feats.shape
    S = H * W
    M = w_gate.shape[1]
    RPB = C * S // 128
    x = feats.reshape(B * RPB, 128)
    BB = 4
    wts = w_gate.T * jnp.float32(1.0 / S)
    r = jnp.arange(RPB)
    u = r // 9
    k = r % 9
    ch_lo = jnp.where(k <= 4, 2 * u, 2 * u + 1)
    ch_hi = jnp.where(k <= 3, 2 * u, 2 * u + 1)
    wlo = jnp.tile(wts[:, ch_lo], (1, BB))
    whi = jnp.tile(wts[:, ch_hi], (1, BB))

    logits = pl.pallas_call(
        functools.partial(_pool_body, BB, RPB),
        grid=(B // BB,),
        in_specs=[
            pl.BlockSpec((BB * RPB, 128), lambda i: (i, 0)),
            pl.BlockSpec((M, BB * RPB), lambda i: (0, 0)),
            pl.BlockSpec((M, BB * RPB), lambda i: (0, 0)),
        ],
        out_specs=pl.BlockSpec((B, M), lambda i: (0, 0)),
        out_shape=jax.ShapeDtypeStruct((B, M), jnp.float32),
    )(x, wlo, whi)

    coef = jnp.reshape(jnp.asarray(loss_coef, jnp.float32), (1,))
    gates, loss = pl.pallas_call(
        _routing_body,
        in_specs=[
            pl.BlockSpec(memory_space=pltpu.VMEM),
            pl.BlockSpec(memory_space=pltpu.SMEM),
        ],
        out_specs=[
            pl.BlockSpec(memory_space=pltpu.VMEM),
            pl.BlockSpec(memory_space=pltpu.SMEM),
        ],
        out_shape=[
            jax.ShapeDtypeStruct((B, M), jnp.float32),
            jax.ShapeDtypeStruct((1, 1), jnp.float32),
        ],
    )(logits, coef)

    return gates, loss[0, 0]


# speedup vs baseline: 17.4893x; 17.4893x over previous
import functools

import jax
import jax.numpy as jnp
from jax.experimental import pallas as pl
from jax.experimental.pallas import tpu as pltpu


def _pool_body(bb, x_ref, wt_ref, o_ref):
    x = x_ref[...]
    ssum = jnp.sum(x, axis=1)
    prod = ssum[:, None, :] * wt_ref[...][None, :, :]
    blk = jnp.sum(prod, axis=2)
    i = pl.program_id(0)
    o_ref[pl.ds(i * bb, bb), :] = blk


def _routing_body(l_ref, coef_ref, g_ref, loss_ref):
    logits = l_ref[...]
    b_dim, m_dim = logits.shape
    col = jax.lax.broadcasted_iota(jnp.int32, (b_dim, m_dim), 1)
    big = jnp.int32(m_dim)

    m1 = jnp.max(logits, axis=1, keepdims=True)
    idx1 = jnp.min(jnp.where(logits == m1, col, big), axis=1, keepdims=True)
    masked = jnp.where(col == idx1, -jnp.inf, logits)
    m2 = jnp.max(masked, axis=1, keepdims=True)
    idx2 = jnp.min(jnp.where(masked == m2, col, big), axis=1, keepdims=True)

    e = jnp.exp(m2 - m1)
    denom = 1.0 + e
    g1 = 1.0 / denom
    g2 = e / denom
    gates = (jnp.where(col == idx1, g1, 0.0)
             + jnp.where(col == idx2, g2, 0.0))
    g_ref[...] = gates

    imp = jnp.sum(gates, axis=0, keepdims=True)
    load = jnp.sum((gates > 0.0).astype(jnp.float32), axis=0,
                   keepdims=True)

    def cv_sq(v):
        mean = jnp.sum(v) * jnp.float32(1.0 / m_dim)
        var = jnp.sum((v - mean) ** 2) * jnp.float32(1.0 / (m_dim - 1))
        return var / (mean * mean + jnp.float32(1e-10))

    loss_ref[0, 0] = (cv_sq(imp) + cv_sq(load)) * coef_ref[0]


def kernel(feats, w_gate, w_noise, loss_coef=0.01, noise_epsilon=0.01):
    B, C, H, W = feats.shape
    S = H * W
    M = w_gate.shape[1]
    x = jnp.transpose(feats, (0, 2, 3, 1)).reshape(B, S, C)
    BB = 4
    wt = w_gate.T * jnp.float32(1.0 / S)

    logits = pl.pallas_call(
        functools.partial(_pool_body, BB),
        grid=(B // BB,),
        in_specs=[
            pl.BlockSpec((BB, S, C), lambda i: (i, 0, 0)),
            pl.BlockSpec((M, C), lambda i: (0, 0)),
        ],
        out_specs=pl.BlockSpec((B, M), lambda i: (0, 0)),
        out_shape=jax.ShapeDtypeStruct((B, M), jnp.float32),
    )(x, wt)

    coef = jnp.reshape(jnp.asarray(loss_coef, jnp.float32), (1,))
    gates, loss = pl.pallas_call(
        _routing_body,
        in_specs=[
            pl.BlockSpec(memory_space=pltpu.VMEM),
            pl.BlockSpec(memory_space=pltpu.SMEM),
        ],
        out_specs=[
            pl.BlockSpec(memory_space=pltpu.VMEM),
            pl.BlockSpec(memory_space=pltpu.SMEM),
        ],
        out_shape=[
            jax.ShapeDtypeStruct((B, M), jnp.float32),
            jax.ShapeDtypeStruct((1, 1), jnp.float32),
        ],
    )(logits, coef)

    return gates, loss[0, 0]


# speedup vs baseline: 18.0621x; 1.0328x over previous
import functools

import jax
import jax.numpy as jnp
from jax.experimental import pallas as pl
from jax.experimental.pallas import tpu as pltpu


def _route(logits, coef):
    b_dim, m_dim = logits.shape
    col = jax.lax.broadcasted_iota(jnp.int32, (b_dim, m_dim), 1)
    big = jnp.int32(m_dim)

    m1 = jnp.max(logits, axis=1, keepdims=True)
    idx1 = jnp.min(jnp.where(logits == m1, col, big), axis=1, keepdims=True)
    masked = jnp.where(col == idx1, -jnp.inf, logits)
    m2 = jnp.max(masked, axis=1, keepdims=True)
    idx2 = jnp.min(jnp.where(masked == m2, col, big), axis=1, keepdims=True)

    e = jnp.exp(m2 - m1)
    denom = 1.0 + e
    g1 = 1.0 / denom
    g2 = e / denom
    gates = (jnp.where(col == idx1, g1, 0.0)
             + jnp.where(col == idx2, g2, 0.0))

    imp = jnp.sum(gates, axis=0, keepdims=True)
    load = jnp.sum((gates > 0.0).astype(jnp.float32), axis=0, keepdims=True)

    def cv_sq(v):
        mean = jnp.sum(v) * jnp.float32(1.0 / m_dim)
        var = jnp.sum((v - mean) ** 2) * jnp.float32(1.0 / (m_dim - 1))
        return var / (mean * mean + jnp.float32(1e-10))

    loss = (cv_sq(imp) + cv_sq(load)) * coef
    return gates, loss


def _fused_body(bb, x_ref, w_ref, coef_ref, g_ref, loss_ref,
                wt_buf, logit_buf):
    i = pl.program_id(0)
    nb = pl.num_programs(0)
    s = x_ref.shape[1]

    @pl.when(i == 0)
    def _prep():
        wt_buf[...] = w_ref[...].T * jnp.float32(1.0 / s)

    x = x_ref[...]
    ssum = jnp.sum(x, axis=1)
    prod = ssum[:, None, :] * wt_buf[...][None, :, :]
    logit_buf[pl.ds(i * bb, bb), :] = jnp.sum(prod, axis=2)

    @pl.when(i == nb - 1)
    def _epilogue():
        gates, loss = _route(logit_buf[...], coef_ref[0])
        g_ref[...] = gates
        loss_ref[0, 0] = loss


def kernel(feats, w_gate, w_noise, loss_coef=0.01, noise_epsilon=0.01):
    B, C, H, W = feats.shape
    S = H * W
    M = w_gate.shape[1]
    x = jnp.transpose(feats, (0, 2, 3, 1)).reshape(B, S, C)
    BB = 4
    coef = jnp.reshape(jnp.asarray(loss_coef, jnp.float32), (1,))

    gates, loss = pl.pallas_call(
        functools.partial(_fused_body, BB),
        grid=(B // BB,),
        in_specs=[
            pl.BlockSpec((BB, S, C), lambda i: (i, 0, 0)),
            pl.BlockSpec((C, M), lambda i: (0, 0)),
            pl.BlockSpec(memory_space=pltpu.SMEM),
        ],
        out_specs=[
            pl.BlockSpec((B, M), lambda i: (0, 0)),
            pl.BlockSpec(memory_space=pltpu.SMEM),
        ],
        out_shape=[
            jax.ShapeDtypeStruct((B, M), jnp.float32),
            jax.ShapeDtypeStruct((1, 1), jnp.float32),
        ],
        scratch_shapes=[
            pltpu.VMEM((M, C), jnp.float32),
            pltpu.VMEM((B, M), jnp.float32),
        ],
    )(x, w_gate, coef)

    return gates, loss[0, 0]


# speedup vs baseline: 18.1128x; 1.0028x over previous
import functools

import jax
import jax.numpy as jnp
from jax.experimental import pallas as pl
from jax.experimental.pallas import tpu as pltpu


def _route(logits, coef):
    b_dim, m_dim = logits.shape
    col = jax.lax.broadcasted_iota(jnp.int32, (b_dim, m_dim), 1)
    big = jnp.int32(m_dim)

    m1 = jnp.max(logits, axis=1, keepdims=True)
    idx1 = jnp.min(jnp.where(logits == m1, col, big), axis=1, keepdims=True)
    masked = jnp.where(col == idx1, -jnp.inf, logits)
    m2 = jnp.max(masked, axis=1, keepdims=True)
    idx2 = jnp.min(jnp.where(masked == m2, col, big), axis=1, keepdims=True)

    e = jnp.exp(m2 - m1)
    denom = 1.0 + e
    g1 = 1.0 / denom
    g2 = e / denom
    gates = (jnp.where(col == idx1, g1, 0.0)
             + jnp.where(col == idx2, g2, 0.0))

    imp = jnp.sum(gates, axis=0, keepdims=True)
    load = jnp.sum((gates > 0.0).astype(jnp.float32), axis=0, keepdims=True)

    def cv_sq(v):
        mean = jnp.sum(v) * jnp.float32(1.0 / m_dim)
        var = jnp.sum((v - mean) ** 2) * jnp.float32(1.0 / (m_dim - 1))
        return var / (mean * mean + jnp.float32(1e-10))

    loss = (cv_sq(imp) + cv_sq(load)) * coef
    return gates, loss


def _fused_body(bb, nsteps, x_ref, w_ref, coef_ref, g_ref, loss_ref,
                wt_buf, logit_buf):
    i = pl.program_id(0)
    s = x_ref.shape[1]

    @pl.when(i == 0)
    def _prep():
        wt_buf[...] = w_ref[...].T * jnp.float32(1.0 / s)

    x = x_ref[...]
    ssum = jnp.sum(x, axis=1)
    prod = ssum[:, None, :] * wt_buf[...][None, :, :]
    blk = jnp.sum(prod, axis=2)

    @pl.when(i < nsteps - 1)
    def _store():
        logit_buf[pl.ds(i * bb, bb), :] = blk

    @pl.when(i == nsteps - 1)
    def _epilogue():
        lg = logit_buf[...]
        last = (nsteps - 1) * bb
        rowi = jax.lax.broadcasted_iota(jnp.int32, lg.shape, 0)
        blk_full = jnp.concatenate(
            [jnp.zeros((last, lg.shape[1]), jnp.float32), blk], axis=0)
        gates, loss = _route(jnp.where(rowi >= last, blk_full, lg),
                             coef_ref[0])
        g_ref[...] = gates
        loss_ref[0, 0] = loss


def kernel(feats, w_gate, w_noise, loss_coef=0.01, noise_epsilon=0.01):
    B, C, H, W = feats.shape
    S = H * W
    M = w_gate.shape[1]
    x = jnp.transpose(feats, (0, 2, 3, 1)).reshape(B, S, C)
    BB = 8
    coef = jnp.reshape(jnp.asarray(loss_coef, jnp.float32), (1,))

    gates, loss = pl.pallas_call(
        functools.partial(_fused_body, BB, B // BB),
        grid=(B // BB,),
        in_specs=[
            pl.BlockSpec((BB, S, C), lambda i: (i, 0, 0)),
            pl.BlockSpec((C, M), lambda i: (0, 0)),
            pl.BlockSpec(memory_space=pltpu.SMEM),
        ],
        out_specs=[
            pl.BlockSpec((B, M), lambda i: (0, 0)),
            pl.BlockSpec(memory_space=pltpu.SMEM),
        ],
        out_shape=[
            jax.ShapeDtypeStruct((B, M), jnp.float32),
            jax.ShapeDtypeStruct((1, 1), jnp.float32),
        ],
        scratch_shapes=[
            pltpu.VMEM((M, C), jnp.float32),
            pltpu.VMEM((B, M), jnp.float32),
        ],
    )(x, w_gate, coef)

    return gates, loss[0, 0]
